# baseline (speedup 1.0000x reference)
import functools

import jax
import jax.numpy as jnp
from jax import lax
from jax.experimental import pallas as pl
from jax.experimental.pallas import tpu as pltpu
from jax.experimental.pallas import tpu_sc as plsc

_L = 16
_NGROUP = 8
_NQ = 4
_CHUNK = 4096

_QLO = (0, 25088, 50176, 75264)


@functools.lru_cache(maxsize=None)
def _build(B, V, D):
    assert D == _NGROUP * 8
    n_full = 6
    tail_a = 25088 - n_full * _CHUNK
    tail_b = (V - _QLO[3]) - n_full * _CHUNK
    mesh = plsc.VectorSubcoreMesh(core_axis_name="c", subcore_axis_name="s")

    out_sds = jax.ShapeDtypeStruct((D, B), jnp.float32)

    @functools.partial(
        pl.kernel,
        mesh=mesh,
        out_type=(out_sds, out_sds, out_sds, out_sds),
        scratch_types=[
            pltpu.VMEM((B,), jnp.int32),
            pltpu.VMEM((B,), jnp.int32),
            pltpu.VMEM((B,), jnp.int32),
            pltpu.VMEM((B,), jnp.int32),
            pltpu.VMEM((B,), jnp.int32),
            pltpu.VMEM((2, 8, _CHUNK), jnp.float32),
            pltpu.VMEM((8, B + _L), jnp.float32),
            pltpu.SemaphoreType.DMA((2,)),
        ],
        compiler_params=pltpu.CompilerParams(needs_layout_passes=False),
    )
    def gather_kernel(idx_hbm, tableT_hbm, o0, o1, o2, o3,
                      idx_v, lid_v, pos_v, clid_v, cpos_v, buf_v, stage_v,
                      sems):
        wid = lax.axis_index("s") * 2 + lax.axis_index("c")
        g = lax.rem(wid, _NGROUP)
        q = wid // _NGROUP
        d0 = pl.multiple_of(g * 8, 8)
        q_lo = jnp.int32(25088) * q
        q_hi = jnp.where(q == _NQ - 1, jnp.int32(V), q_lo + 25088)
        v0 = pl.multiple_of(q_lo, 128)

        pltpu.sync_copy(idx_hbm, idx_v)

        iota = lax.iota(jnp.int32, _L)

        def pre_body(i, cnt):
            ids16 = idx_v[pl.ds(i * _L, _L)]
            m = (ids16 >= q_lo) & (ids16 < q_hi)
            plsc.store_compressed(lid_v.at[pl.ds(cnt, _L)], ids16 - v0, mask=m)
            plsc.store_compressed(pos_v.at[pl.ds(cnt, _L)], i * _L + iota, mask=m)
            return cnt + plsc.all_reduce_population_count(m)[0]

        cnt = lax.fori_loop(0, B // _L, pre_body, jnp.int32(0))

        def start_chunk(c, slot):
            pltpu.make_async_copy(
                tableT_hbm.at[pl.ds(d0, 8), pl.ds(v0 + c * _CHUNK, _CHUNK)],
                buf_v.at[slot], sems.at[slot]).start()

        def wait_chunk(c, slot):
            pltpu.make_async_copy(
                tableT_hbm.at[pl.ds(d0, 8), pl.ds(v0 + c * _CHUNK, _CHUNK)],
                buf_v.at[slot], sems.at[slot]).wait()

        start_chunk(0, 0)

        def process(c_lo, c_hi, slot, cnt):
            def re_body(j, ccnt):
                off = j * _L
                lane_ok = iota < (cnt - off)
                l16 = lid_v[pl.ds(off, _L)]
                p16 = pos_v[pl.ds(off, _L)]
                m = (l16 >= c_lo) & (l16 < c_hi) & lane_ok
                plsc.store_compressed(clid_v.at[pl.ds(ccnt, _L)], l16 - c_lo, mask=m)
                plsc.store_compressed(cpos_v.at[pl.ds(ccnt, _L)], p16, mask=m)
                return ccnt + plsc.all_reduce_population_count(m)[0]

            ccnt = lax.fori_loop(0, (cnt + _L - 1) // _L, re_body,
                                 jnp.int32(0))

            def hit_body(j, carry):
                off = j * _L
                lane_ok = iota < (ccnt - off)
                vl = jnp.where(lane_ok, clid_v[pl.ds(off, _L)], 0)
                p16 = jnp.where(lane_ok, cpos_v[pl.ds(off, _L)], B + iota)
                for d in range(8):
                    dsplat = jnp.full((_L,), d, jnp.int32)
                    vals = plsc.load_gather(buf_v.at[slot], [dsplat, vl])
                    plsc.store_scatter(stage_v, [dsplat, p16], vals)
                return carry

            lax.fori_loop(0, (ccnt + _L - 1) // _L, hit_body, jnp.int32(0))

        def chunk_body(c, cnt):
            slot = lax.rem(c, 2)

            @pl.when(c + 1 < n_full)
            def _():
                start_chunk(c + 1, 1 - slot)

            wait_chunk(c, slot)
            process(c * _CHUNK, (c + 1) * _CHUNK, slot, cnt)
            return cnt

        lax.fori_loop(0, n_full, chunk_body, cnt)

        t_lo = jnp.int32(n_full * _CHUNK)

        @pl.when(q < _NQ - 1)
        def _():
            pltpu.sync_copy(
                tableT_hbm.at[pl.ds(d0, 8),
                              pl.ds(v0 + n_full * _CHUNK, tail_a)],
                buf_v.at[0, :, pl.ds(0, tail_a)])
            process(t_lo, t_lo + tail_a, 0, cnt)

        @pl.when(q == _NQ - 1)
        def _():
            pltpu.sync_copy(
                tableT_hbm.at[pl.ds(d0, 8),
                              pl.ds(v0 + n_full * _CHUNK, 256)],
                buf_v.at[0, :, pl.ds(0, 256)])
            process(t_lo, t_lo + tail_b, 0, cnt)

        for qq, o in enumerate((o0, o1, o2, o3)):
            @pl.when(q == qq)
            def _(o=o):
                pltpu.sync_copy(stage_v.at[:, pl.ds(0, B)], o.at[pl.ds(d0, 8)])

    return gather_kernel


def kernel(node_ids, table):
    B = node_ids.shape[0]
    V, D = table.shape
    ids = node_ids.astype(jnp.int32)
    o0, o1, o2, o3 = _build(B, V, D)(ids, table.T)
    b = ids[None, :]
    outT = jnp.where(
        b < 25088, o0,
        jnp.where(b < 50176, o1, jnp.where(b < 75264, o2, o3)))
    return outT.T


# speedup vs baseline: 1.0479x; 1.0479x over previous
import functools

import jax
import jax.numpy as jnp
from jax import lax
from jax.experimental import pallas as pl
from jax.experimental.pallas import tpu as pltpu
from jax.experimental.pallas import tpu_sc as plsc

_L = 16
_NGROUP = 8
_NQ = 4
_CHUNK = 4096

_QLO = (0, 25088, 50176, 75264)


@functools.lru_cache(maxsize=None)
def _build(B, V, D):
    assert D == _NGROUP * 8
    n_full = 6
    tail_a = 25088 - n_full * _CHUNK
    tail_b = (V - _QLO[3]) - n_full * _CHUNK
    mesh = plsc.VectorSubcoreMesh(core_axis_name="c", subcore_axis_name="s")

    out_sds = jax.ShapeDtypeStruct((D, B), jnp.float32)

    @functools.partial(
        pl.kernel,
        mesh=mesh,
        out_type=(out_sds, out_sds, out_sds, out_sds),
        scratch_types=[
            pltpu.VMEM((B,), jnp.int32),
            pltpu.VMEM((B,), jnp.int32),
            pltpu.VMEM((B,), jnp.int32),
            pltpu.VMEM((B,), jnp.int32),
            pltpu.VMEM((B,), jnp.int32),
            pltpu.VMEM((2, 8, _CHUNK), jnp.float32),
            pltpu.VMEM((8, B + _L), jnp.float32),
            pltpu.SemaphoreType.DMA((2,)),
        ],
        compiler_params=pltpu.CompilerParams(needs_layout_passes=False),
    )
    def gather_kernel(idx_hbm, tableT_hbm, o0, o1, o2, o3,
                      idx_v, lid_v, pos_v, clid_v, cpos_v, buf_v, stage_v,
                      sems):
        wid = lax.axis_index("s") * 2 + lax.axis_index("c")
        g = lax.rem(wid, _NGROUP)
        q = wid // _NGROUP
        d0 = pl.multiple_of(g * 8, 8)
        q_lo = jnp.int32(25088) * q
        q_hi = jnp.where(q == _NQ - 1, jnp.int32(V), q_lo + 25088)
        v0 = pl.multiple_of(q_lo, 128)

        iota = lax.iota(jnp.int32, _L)

        def start_chunk(c, slot):
            pltpu.make_async_copy(
                tableT_hbm.at[pl.ds(d0, 8), pl.ds(v0 + c * _CHUNK, _CHUNK)],
                buf_v.at[slot], sems.at[slot]).start()

        def wait_chunk(c, slot):
            pltpu.make_async_copy(
                tableT_hbm.at[pl.ds(d0, 8), pl.ds(v0 + c * _CHUNK, _CHUNK)],
                buf_v.at[slot], sems.at[slot]).wait()

        start_chunk(0, 0)
        start_chunk(1, 1)
        pltpu.sync_copy(idx_hbm, idx_v)

        def pre_body(i, cnt):
            for u in range(4):
                ii = i * 4 + u
                ids16 = idx_v[pl.ds(ii * _L, _L)]
                m = (ids16 >= q_lo) & (ids16 < q_hi)
                plsc.store_compressed(lid_v.at[pl.ds(cnt, _L)], ids16 - v0,
                                      mask=m)
                plsc.store_compressed(pos_v.at[pl.ds(cnt, _L)],
                                      ii * _L + iota, mask=m)
                cnt = cnt + plsc.all_reduce_population_count(m)[0]
            return cnt

        cnt = lax.fori_loop(0, B // _L // 4, pre_body, jnp.int32(0))

        def process(c_lo, c_hi, slot, cnt):
            def re_body(j, ccnt):
                for u in range(4):
                    off = (j * 4 + u) * _L
                    lane_ok = iota < (cnt - off)
                    l16 = lid_v[pl.ds(off, _L)]
                    p16 = pos_v[pl.ds(off, _L)]
                    m = (l16 >= c_lo) & (l16 < c_hi) & lane_ok
                    plsc.store_compressed(clid_v.at[pl.ds(ccnt, _L)],
                                          l16 - c_lo, mask=m)
                    plsc.store_compressed(cpos_v.at[pl.ds(ccnt, _L)], p16,
                                          mask=m)
                    ccnt = ccnt + plsc.all_reduce_population_count(m)[0]
                return ccnt

            ccnt = lax.fori_loop(0, (cnt + 4 * _L - 1) // (4 * _L), re_body,
                                 jnp.int32(0))

            def hit_body(j, carry):
                off = j * _L
                lane_ok = iota < (ccnt - off)
                vl = jnp.where(lane_ok, clid_v[pl.ds(off, _L)], 0)
                p16 = jnp.where(lane_ok, cpos_v[pl.ds(off, _L)], B + iota)
                for d in range(8):
                    dsplat = jnp.full((_L,), d, jnp.int32)
                    vals = plsc.load_gather(buf_v.at[slot], [dsplat, vl])
                    plsc.store_scatter(stage_v, [dsplat, p16], vals)
                return carry

            lax.fori_loop(0, (ccnt + _L - 1) // _L, hit_body, jnp.int32(0))

        def chunk_body(c, cnt):
            slot = lax.rem(c, 2)
            wait_chunk(c, slot)
            process(c * _CHUNK, (c + 1) * _CHUNK, slot, cnt)

            @pl.when(c + 2 < n_full)
            def _():
                start_chunk(c + 2, slot)

            return cnt

        lax.fori_loop(0, n_full, chunk_body, cnt)

        t_lo = jnp.int32(n_full * _CHUNK)

        @pl.when(q < _NQ - 1)
        def _():
            pltpu.sync_copy(
                tableT_hbm.at[pl.ds(d0, 8),
                              pl.ds(v0 + n_full * _CHUNK, tail_a)],
                buf_v.at[0, :, pl.ds(0, tail_a)])
            process(t_lo, t_lo + tail_a, 0, cnt)

        @pl.when(q == _NQ - 1)
        def _():
            pltpu.sync_copy(
                tableT_hbm.at[pl.ds(d0, 8),
                              pl.ds(v0 + n_full * _CHUNK, 256)],
                buf_v.at[0, :, pl.ds(0, 256)])
            process(t_lo, t_lo + tail_b, 0, cnt)

        for qq, o in enumerate((o0, o1, o2, o3)):
            @pl.when(q == qq)
            def _(o=o):
                pltpu.sync_copy(stage_v.at[:, pl.ds(0, B)], o.at[pl.ds(d0, 8)])

    return gather_kernel


def kernel(node_ids, table):
    B = node_ids.shape[0]
    V, D = table.shape
    ids = node_ids.astype(jnp.int32)
    o0, o1, o2, o3 = _build(B, V, D)(ids, table.T)
    b = ids[None, :]
    outT = jnp.where(
        b < 25088, o0,
        jnp.where(b < 50176, o1, jnp.where(b < 75264, o2, o3)))
    return outT.T
